# baseline (speedup 1.0000x reference)
import jax
import jax.numpy as jnp
from jax import lax
from jax.experimental import pallas as pl
from jax.experimental.pallas import tpu as pltpu
from jax.experimental.pallas import tpu_sc as plsc

N_AUX = 500000
N_TOT = 1000000
NW = 32
CH_A = 15632
CH_A_TAIL = N_AUX - (NW - 1) * CH_A

NB = 8
HALF = 524288
HI_SZ = N_TOT - HALF
DUMP = HALF
WSH = HALF + 2048
CAP = 2560
BLK = 16 * CAP
SC_SEG = NB * BLK
SC_SEGD = SC_SEG + 2048
SEGSZ = 2 * SC_SEG
GSZ = 4 * CAP
SUB0 = 7808
SUB1 = CH_A - SUB0
BOUNCE = SC_SEG // 16 // 2
FILL = 16384
BIG = 2 ** 30


def _vmin(v):
    iota = lax.iota(jnp.int32, 16)
    for s in (8, 4, 2, 1):
        v = jnp.minimum(v, v.at[iota ^ s].get(mode="promise_in_bounds"))
    return v


def _mesh():
    return plsc.VectorSubcoreMesh(core_axis_name="c", subcore_axis_name="s",
                                  num_cores=2, num_subcores=16)


def _stage1(index_hbm, tidx_hbm, t16_hbm, iseg_hbm, vseg_hbm,
            pdiff_hbm, ppos_hbm, *scr):
    (xch, tch, tv, rdv, rpv, dmpb, ic0, ic1, mc0, mc1, sb0, sb1,
     ist, vst, sem_f, sem_s) = scr
    cid = lax.axis_index("c")
    sid = lax.axis_index("s")
    wid = cid * 16 + sid
    pltpu.sync_copy(t16_hbm, tv)
    t = tv[...]
    iota = lax.iota(jnp.int32, 16)

    shift_idx = [jnp.maximum(iota - s, 0) for s in (1, 2, 4, 8)]
    shift_msk = [iota >= s for s in (1, 2, 4, 8)]
    c15 = jnp.full((16,), 15, jnp.int32)
    zero = jnp.zeros((16,), jnp.int32)

    def prefix(v):
        for gi, gm in zip(shift_idx, shift_msk):
            g = v.at[gi].get(mode="promise_in_bounds")
            v = v + jnp.where(gm, g, zero)
        return v

    tot_sh = (iota & 3) << 3
    lo_lane = iota < 4

    def work(n):
        base = wid * CH_A
        pltpu.sync_copy(index_hbm.at[pl.ds(base, n)], xch.at[pl.ds(0, n)])
        pltpu.sync_copy(tidx_hbm.at[pl.ds(base, n)], tch.at[pl.ds(0, n)])

        def db(i, z):
            r = i // (CAP // 16)
            k = i - r * (CAP // 16)
            dmpb[pl.ds(i * 16, 16)] = DUMP + (r << 9) + ((k * 16 + iota) & 511)
            return z

        lax.fori_loop(0, 4 * CAP // 16, db, 0)
        fills = [pltpu.async_copy(
            dmpb.at[pl.ds((p % 4) * CAP, CAP)],
            ist.at[pl.ds(p * BLK + sid * CAP, CAP)], sem_f)
            for p in range(NB)]

        def body(j0, ic, mc, sb):
            def f(j, carry):
                bestd, bestp, hv = carry
                x = xch[pl.ds((j0 + j) * 16, 16)]
                m = 1.0 / (1.0 + jnp.exp((x - t) * 10.0))
                d = jnp.abs(x - t)
                pos = base + (j0 + j) * 16 + iota
                take = d < bestd
                bestd = jnp.where(take, d, bestd)
                bestp = jnp.where(take, pos, bestp)
                idx = tch[pl.ds((j0 + j) * 16, 16)]
                bkt = lax.shift_right_logical(idx, 17)
                hi = bkt >= 4
                idxl = idx - jnp.where(hi, jnp.int32(HALF), 0)
                sh = (bkt & 3) << 3
                onep = jnp.left_shift(jnp.int32(1), sh)
                one_a = jnp.where(hi, zero, onep)
                one_b = jnp.where(hi, onep, zero)
                inc_a = prefix(one_a)
                inc_b = prefix(one_b)
                ex = jnp.where(hi, inc_b - one_b, inc_a - one_a)
                rank = lax.shift_right_logical(ex, sh) & 255
                tot_a = inc_a.at[c15].get(mode="promise_in_bounds")
                tot_b = inc_b.at[c15].get(mode="promise_in_bounds")
                tv_ = jnp.where(lo_lane, tot_a, tot_b)
                tv_ = lax.shift_right_logical(tv_, tot_sh) & 255
                slot = hv.at[bkt].get(mode="promise_in_bounds") + rank
                ic[pl.ds(j * 16, 16)] = idxl
                mc[pl.ds(j * 16, 16)] = m
                sb[pl.ds(j * 16, 16)] = slot
                return (bestd, bestp, hv + tv_)

            return f

        init = (jnp.full((16,), jnp.inf, jnp.float32),
                jnp.zeros((16,), jnp.int32),
                iota * BLK + sid * CAP)
        n0 = SUB0 // 16
        carry = lax.fori_loop(0, n0, body(0, ic0, mc0, sb0), init)
        for d in fills:
            d.wait()
        s0a = pltpu.async_copy(ic0, ist.at[sb0], sem_s)
        s0b = pltpu.async_copy(mc0, vst.at[sb0], sem_s)
        n1 = (n - SUB0) // 16
        bestd, bestp, hv = lax.fori_loop(0, n1, body(n0, ic1, mc1, sb1), carry)
        if n < CH_A:
            for i in range(n1, SUB1 // 16):
                sb1[pl.ds(i * 16, 16)] = SC_SEG + ((i * 16 + iota) & 2047)
        s1a = pltpu.async_copy(ic1, ist.at[sb1], sem_s)
        s1b = pltpu.async_copy(mc1, vst.at[sb1], sem_s)

        dmin = _vmin(bestd)
        pmin = _vmin(jnp.where(bestd == dmin, bestp, jnp.int32(BIG)))
        rdv[...] = dmin
        rpv[...] = pmin
        pltpu.sync_copy(rdv, pdiff_hbm.at[wid])
        pltpu.sync_copy(rpv, ppos_hbm.at[wid])
        s0a.wait()
        s0b.wait()
        s1a.wait()
        s1b.wait()

        plsc.subcore_barrier()
        loff = sid * (SC_SEG // 16)
        for h in range(2):
            o = loff + h * BOUNCE
            pltpu.sync_copy(ist.at[pl.ds(o, BOUNCE)], tch.at[pl.ds(0, BOUNCE)])
            pltpu.sync_copy(tch.at[pl.ds(0, BOUNCE)],
                            iseg_hbm.at[pl.ds(cid * SC_SEG + o, BOUNCE)])
            pltpu.sync_copy(vst.at[pl.ds(o, BOUNCE)], xch.at[pl.ds(0, BOUNCE)])
            pltpu.sync_copy(xch.at[pl.ds(0, BOUNCE)],
                            vseg_hbm.at[pl.ds(cid * SC_SEG + o, BOUNCE)])

    @pl.when(wid < NW - 1)
    def _():
        work(CH_A)

    @pl.when(wid == NW - 1)
    def _():
        work(CH_A_TAIL)


def _stage2(iseg_hbm, vseg_hbm, pdiff_hbm, ppos_hbm, un_hbm,
            w_hbm, k_hbm, *scr):
    wsh, fillv, ib0, ib1, vb0, vb1, pdv, ppv, posv, kv = scr[:10]
    sem_g, sem_s, sem_k = scr[10:13]
    cid = lax.axis_index("c")
    sid = lax.axis_index("s")
    wid = cid * 16 + sid

    def fb(i, z):
        fillv[pl.ds(i * 16, 16)] = jnp.full((16,), 1.0, jnp.float32)
        return z

    lax.fori_loop(0, FILL // 16, fb, 0)
    fbase = sid * (WSH // 16)
    pltpu.sync_copy(fillv, wsh.at[pl.ds(fbase, FILL)])
    pltpu.sync_copy(fillv, wsh.at[pl.ds(fbase + FILL, FILL)])
    pltpu.sync_copy(fillv.at[pl.ds(0, WSH // 16 - 2 * FILL)],
                    wsh.at[pl.ds(fbase + 2 * FILL, WSH // 16 - 2 * FILL)])
    plsc.subcore_barrier()

    @pl.when(sid % 4 == 0)
    def _():
        p = cid * 4 + sid // 4
        ibs = (ib0, ib1)
        vbs = (vb0, vb1)

        NG = 2 * BLK // GSZ

        def start_gather(g, b):
            off = (g // 4) * SC_SEG + p * BLK + (g % 4) * GSZ
            d1 = pltpu.async_copy(iseg_hbm.at[pl.ds(off, GSZ)], ibs[b], sem_g)
            d2 = pltpu.async_copy(vseg_hbm.at[pl.ds(off, GSZ)], vbs[b], sem_g)
            return (d1, d2)

        pending = start_gather(0, 0)
        for g in range(NG):
            b = g % 2
            pending[0].wait()
            pending[1].wait()
            if g + 1 < NG:
                pending = start_gather(g + 1, 1 - b)
            pltpu.async_copy(vbs[b], wsh.at[ibs[b]], sem_s).wait()

    plsc.subcore_barrier()

    def bounce(loff, hoff, n):
        pltpu.sync_copy(wsh.at[pl.ds(loff, n)], fillv.at[pl.ds(0, n)])
        pltpu.sync_copy(fillv.at[pl.ds(0, n)], w_hbm.at[pl.ds(hoff, n)])

    @pl.when(cid == 0)
    def _():
        base = sid * (HALF // 16)
        bounce(base, base, FILL)
        bounce(base + FILL, base + FILL, FILL)

    @pl.when(cid == 1)
    def _():
        tsz = 29728

        @pl.when(sid < 15)
        def _():
            base = sid * tsz
            bounce(base, HALF + base, FILL)
            bounce(base + FILL, HALF + base + FILL, tsz - FILL)

        @pl.when(sid == 15)
        def _():
            base = 15 * tsz
            rem = HI_SZ - base
            bounce(base, HALF + base, FILL)
            bounce(base + FILL, HALF + base + FILL, rem - FILL)

    @pl.when(wid == 0)
    def _():
        pltpu.sync_copy(pdiff_hbm, pdv)
        pltpu.sync_copy(ppos_hbm, ppv)

        def rb(r, carry):
            bd, bp = carry
            d = pdv[r][0]
            pp = ppv[r][0]
            take = (d < bd) | ((d == bd) & (pp < bp))
            return (jnp.where(take, d, bd), jnp.where(take, pp, bp))

        bd, bp = lax.fori_loop(0, NW, rb,
                               (jnp.float32(jnp.inf), jnp.int32(BIG)))
        posv[...] = jnp.full((16,), bp, jnp.int32)
        pltpu.async_copy(un_hbm.at[posv], kv, sem_k).wait()
        pltpu.sync_copy(kv, k_hbm)


def kernel(y, eval_gene_idx, train_highly_gene_idx, index, unnorm_index, thresh):
    t16 = jnp.broadcast_to(jnp.asarray(thresh, jnp.float32), (16,))

    k1 = pl.kernel(
        _stage1,
        out_type=(
            jax.ShapeDtypeStruct((SEGSZ,), jnp.int32),
            jax.ShapeDtypeStruct((SEGSZ,), jnp.float32),
            jax.ShapeDtypeStruct((NW, 16), jnp.float32),
            jax.ShapeDtypeStruct((NW, 16), jnp.int32),
        ),
        mesh=_mesh(),
        scratch_types=(
            pltpu.VMEM((CH_A,), jnp.float32),
            pltpu.VMEM((CH_A,), jnp.int32),
            pltpu.VMEM((16,), jnp.float32),
            pltpu.VMEM((16,), jnp.float32),
            pltpu.VMEM((16,), jnp.int32),
            pltpu.VMEM((4 * CAP,), jnp.int32),
            pltpu.VMEM((SUB0,), jnp.int32),
            pltpu.VMEM((SUB1,), jnp.int32),
            pltpu.VMEM((SUB0,), jnp.float32),
            pltpu.VMEM((SUB1,), jnp.float32),
            pltpu.VMEM((SUB0,), jnp.int32),
            pltpu.VMEM((SUB1,), jnp.int32),
            pltpu.VMEM_SHARED((SC_SEGD,), jnp.int32),
            pltpu.VMEM_SHARED((SC_SEGD,), jnp.float32),
            pltpu.SemaphoreType.DMA,
            pltpu.SemaphoreType.DMA,
        ),
    )
    iseg, vseg, pdiff, ppos = k1(index, train_highly_gene_idx, t16)

    k2 = pl.kernel(
        _stage2,
        out_type=(
            jax.ShapeDtypeStruct((N_TOT,), jnp.float32),
            jax.ShapeDtypeStruct((16,), jnp.int32),
        ),
        mesh=_mesh(),
        scratch_types=(
            pltpu.VMEM_SHARED((WSH,), jnp.float32),
            pltpu.VMEM((FILL,), jnp.float32),
            pltpu.VMEM((GSZ,), jnp.int32),
            pltpu.VMEM((GSZ,), jnp.int32),
            pltpu.VMEM((GSZ,), jnp.float32),
            pltpu.VMEM((GSZ,), jnp.float32),
            pltpu.VMEM((NW, 16), jnp.float32),
            pltpu.VMEM((NW, 16), jnp.int32),
            pltpu.VMEM((16,), jnp.int32),
            pltpu.VMEM((16,), jnp.int32),
            pltpu.SemaphoreType.DMA,
            pltpu.SemaphoreType.DMA,
            pltpu.SemaphoreType.DMA,
        ),
    )
    w, k16 = k2(iseg, vseg, pdiff, ppos, unnorm_index)
    return (w, w, thresh, k16[0])


# speedup vs baseline: 1.0183x; 1.0183x over previous
import jax
import jax.numpy as jnp
from jax import lax
from jax.experimental import pallas as pl
from jax.experimental.pallas import tpu as pltpu
from jax.experimental.pallas import tpu_sc as plsc

N_AUX = 500000
N_TOT = 1000000
NW = 32
CH_A = 15632
CH_A_TAIL = N_AUX - (NW - 1) * CH_A

NB = 8
HALF = 524288
HI_SZ = N_TOT - HALF
DUMP = HALF
WSH = HALF + 2048
CAP = 2560
BLK = 16 * CAP
SC_SEG = NB * BLK
SC_SEGD = SC_SEG + 2048
SEGSZ = 2 * SC_SEG
GSZ = 4 * CAP
SUB0 = 7808
SUB1 = CH_A - SUB0
BOUNCE = SC_SEG // 16 // 2
FILL = 16384
BIG = 2 ** 30


def _vmin(v):
    iota = lax.iota(jnp.int32, 16)
    for s in (8, 4, 2, 1):
        v = jnp.minimum(v, v.at[iota ^ s].get(mode="promise_in_bounds"))
    return v


def _mesh():
    return plsc.VectorSubcoreMesh(core_axis_name="c", subcore_axis_name="s",
                                  num_cores=2, num_subcores=16)


def _stage1(index_hbm, tidx_hbm, t16_hbm, iseg_hbm, vseg_hbm,
            pdiff_hbm, ppos_hbm, *scr):
    (xch, tch, tv, rdv, rpv, dmpb, ic0, ic1, mc0, mc1, sb0, sb1,
     ist, vst, sem_f, sem_s) = scr
    cid = lax.axis_index("c")
    sid = lax.axis_index("s")
    wid = cid * 16 + sid
    pltpu.sync_copy(t16_hbm, tv)
    t = tv[...]
    iota = lax.iota(jnp.int32, 16)

    shift_idx = [jnp.maximum(iota - s, 0) for s in (1, 2, 4, 8)]
    shift_msk = [iota >= s for s in (1, 2, 4, 8)]
    c15 = jnp.full((16,), 15, jnp.int32)
    zero = jnp.zeros((16,), jnp.int32)

    def prefix(v):
        for gi, gm in zip(shift_idx, shift_msk):
            g = v.at[gi].get(mode="promise_in_bounds")
            v = v + jnp.where(gm, g, zero)
        return v

    tot_sh = (iota & 3) << 3
    lo_lane = iota < 4

    def work(n):
        base = wid * CH_A
        pltpu.sync_copy(index_hbm.at[pl.ds(base, n)], xch.at[pl.ds(0, n)])
        pltpu.sync_copy(tidx_hbm.at[pl.ds(base, n)], tch.at[pl.ds(0, n)])

        def db(i, z):
            dmpb[pl.ds(i * 16, 16)] = DUMP + ((i * 16 + iota) & 2047)
            return z

        lax.fori_loop(0, CAP // 16, db, 0)
        fills = [pltpu.async_copy(
            dmpb, ist.at[pl.ds(p * BLK + sid * CAP, CAP)], sem_f)
            for p in range(NB)]

        def body(j0, ic, mc, sb):
            def f(j, carry):
                bestd, bestp, hv = carry
                x = xch[pl.ds((j0 + j) * 16, 16)]
                m = 1.0 / (1.0 + jnp.exp((x - t) * 10.0))
                d = jnp.abs(x - t)
                pos = base + (j0 + j) * 16 + iota
                take = d < bestd
                bestd = jnp.where(take, d, bestd)
                bestp = jnp.where(take, pos, bestp)
                idx = tch[pl.ds((j0 + j) * 16, 16)]
                bkt = lax.shift_right_logical(idx, 17)
                hi = bkt >= 4
                idxl = idx - jnp.where(hi, jnp.int32(HALF), 0)
                sh = (bkt & 3) << 3
                onep = jnp.left_shift(jnp.int32(1), sh)
                one_a = jnp.where(hi, zero, onep)
                one_b = jnp.where(hi, onep, zero)
                inc_a = prefix(one_a)
                inc_b = prefix(one_b)
                ex = jnp.where(hi, inc_b - one_b, inc_a - one_a)
                rank = lax.shift_right_logical(ex, sh) & 255
                tot_a = inc_a.at[c15].get(mode="promise_in_bounds")
                tot_b = inc_b.at[c15].get(mode="promise_in_bounds")
                tv_ = jnp.where(lo_lane, tot_a, tot_b)
                tv_ = lax.shift_right_logical(tv_, tot_sh) & 255
                slot = hv.at[bkt].get(mode="promise_in_bounds") + rank
                ic[pl.ds(j * 16, 16)] = idxl
                mc[pl.ds(j * 16, 16)] = m
                sb[pl.ds(j * 16, 16)] = slot
                return (bestd, bestp, hv + tv_)

            return f

        init = (jnp.full((16,), jnp.inf, jnp.float32),
                jnp.zeros((16,), jnp.int32),
                iota * BLK + sid * CAP)
        n0 = SUB0 // 16
        carry = lax.fori_loop(0, n0, body(0, ic0, mc0, sb0), init)
        for d in fills:
            d.wait()
        s0a = pltpu.async_copy(ic0, ist.at[sb0], sem_s)
        s0b = pltpu.async_copy(mc0, vst.at[sb0], sem_s)
        n1 = (n - SUB0) // 16
        bestd, bestp, hv = lax.fori_loop(0, n1, body(n0, ic1, mc1, sb1), carry)
        if n < CH_A:
            for i in range(n1, SUB1 // 16):
                sb1[pl.ds(i * 16, 16)] = SC_SEG + ((i * 16 + iota) & 2047)
        s1a = pltpu.async_copy(ic1, ist.at[sb1], sem_s)
        s1b = pltpu.async_copy(mc1, vst.at[sb1], sem_s)

        dmin = _vmin(bestd)
        pmin = _vmin(jnp.where(bestd == dmin, bestp, jnp.int32(BIG)))
        rdv[...] = dmin
        rpv[...] = pmin
        pltpu.sync_copy(rdv, pdiff_hbm.at[wid])
        pltpu.sync_copy(rpv, ppos_hbm.at[wid])
        s0a.wait()
        s0b.wait()
        s1a.wait()
        s1b.wait()

        plsc.subcore_barrier()
        loff = sid * (SC_SEG // 16)
        for h in range(2):
            o = loff + h * BOUNCE
            pltpu.sync_copy(ist.at[pl.ds(o, BOUNCE)], tch.at[pl.ds(0, BOUNCE)])
            pltpu.sync_copy(tch.at[pl.ds(0, BOUNCE)],
                            iseg_hbm.at[pl.ds(cid * SC_SEG + o, BOUNCE)])
            pltpu.sync_copy(vst.at[pl.ds(o, BOUNCE)], xch.at[pl.ds(0, BOUNCE)])
            pltpu.sync_copy(xch.at[pl.ds(0, BOUNCE)],
                            vseg_hbm.at[pl.ds(cid * SC_SEG + o, BOUNCE)])

    @pl.when(wid < NW - 1)
    def _():
        work(CH_A)

    @pl.when(wid == NW - 1)
    def _():
        work(CH_A_TAIL)


def _stage2(iseg_hbm, vseg_hbm, pdiff_hbm, ppos_hbm, un_hbm,
            w_hbm, k_hbm, *scr):
    wsh, fillv, ib0, ib1, vb0, vb1, pdv, ppv, posv, kv = scr[:10]
    sem_g, sem_s, sem_k = scr[10:13]
    cid = lax.axis_index("c")
    sid = lax.axis_index("s")
    wid = cid * 16 + sid

    def fb(i, z):
        fillv[pl.ds(i * 16, 16)] = jnp.full((16,), 1.0, jnp.float32)
        return z

    lax.fori_loop(0, FILL // 16, fb, 0)
    fbase = sid * (WSH // 16)
    pltpu.sync_copy(fillv, wsh.at[pl.ds(fbase, FILL)])
    pltpu.sync_copy(fillv, wsh.at[pl.ds(fbase + FILL, FILL)])
    pltpu.sync_copy(fillv.at[pl.ds(0, WSH // 16 - 2 * FILL)],
                    wsh.at[pl.ds(fbase + 2 * FILL, WSH // 16 - 2 * FILL)])
    plsc.subcore_barrier()

    @pl.when(sid % 4 == 0)
    def _():
        p = cid * 4 + sid // 4
        ibs = (ib0, ib1)
        vbs = (vb0, vb1)

        NG = 2 * BLK // GSZ

        def start_gather(g, b):
            off = (g // 4) * SC_SEG + p * BLK + (g % 4) * GSZ
            d1 = pltpu.async_copy(iseg_hbm.at[pl.ds(off, GSZ)], ibs[b], sem_g)
            d2 = pltpu.async_copy(vseg_hbm.at[pl.ds(off, GSZ)], vbs[b], sem_g)
            return (d1, d2)

        pending = start_gather(0, 0)
        for g in range(NG):
            b = g % 2
            pending[0].wait()
            pending[1].wait()
            if g + 1 < NG:
                pending = start_gather(g + 1, 1 - b)
            pltpu.async_copy(vbs[b], wsh.at[ibs[b]], sem_s).wait()

    plsc.subcore_barrier()

    def bounce(loff, hoff, n):
        pltpu.sync_copy(wsh.at[pl.ds(loff, n)], fillv.at[pl.ds(0, n)])
        pltpu.sync_copy(fillv.at[pl.ds(0, n)], w_hbm.at[pl.ds(hoff, n)])

    @pl.when(cid == 0)
    def _():
        base = sid * (HALF // 16)
        bounce(base, base, FILL)
        bounce(base + FILL, base + FILL, FILL)

    @pl.when(cid == 1)
    def _():
        tsz = 29728

        @pl.when(sid < 15)
        def _():
            base = sid * tsz
            bounce(base, HALF + base, FILL)
            bounce(base + FILL, HALF + base + FILL, tsz - FILL)

        @pl.when(sid == 15)
        def _():
            base = 15 * tsz
            rem = HI_SZ - base
            bounce(base, HALF + base, FILL)
            bounce(base + FILL, HALF + base + FILL, rem - FILL)

    @pl.when(wid == 0)
    def _():
        pltpu.sync_copy(pdiff_hbm, pdv)
        pltpu.sync_copy(ppos_hbm, ppv)

        def rb(r, carry):
            bd, bp = carry
            d = pdv[r][0]
            pp = ppv[r][0]
            take = (d < bd) | ((d == bd) & (pp < bp))
            return (jnp.where(take, d, bd), jnp.where(take, pp, bp))

        bd, bp = lax.fori_loop(0, NW, rb,
                               (jnp.float32(jnp.inf), jnp.int32(BIG)))
        posv[...] = jnp.full((16,), bp, jnp.int32)
        pltpu.async_copy(un_hbm.at[posv], kv, sem_k).wait()
        pltpu.sync_copy(kv, k_hbm)


def kernel(y, eval_gene_idx, train_highly_gene_idx, index, unnorm_index, thresh):
    t16 = jnp.broadcast_to(jnp.asarray(thresh, jnp.float32), (16,))

    k1 = pl.kernel(
        _stage1,
        out_type=(
            jax.ShapeDtypeStruct((SEGSZ,), jnp.int32),
            jax.ShapeDtypeStruct((SEGSZ,), jnp.float32),
            jax.ShapeDtypeStruct((NW, 16), jnp.float32),
            jax.ShapeDtypeStruct((NW, 16), jnp.int32),
        ),
        mesh=_mesh(),
        scratch_types=(
            pltpu.VMEM((CH_A,), jnp.float32),
            pltpu.VMEM((CH_A,), jnp.int32),
            pltpu.VMEM((16,), jnp.float32),
            pltpu.VMEM((16,), jnp.float32),
            pltpu.VMEM((16,), jnp.int32),
            pltpu.VMEM((CAP,), jnp.int32),
            pltpu.VMEM((SUB0,), jnp.int32),
            pltpu.VMEM((SUB1,), jnp.int32),
            pltpu.VMEM((SUB0,), jnp.float32),
            pltpu.VMEM((SUB1,), jnp.float32),
            pltpu.VMEM((SUB0,), jnp.int32),
            pltpu.VMEM((SUB1,), jnp.int32),
            pltpu.VMEM_SHARED((SC_SEGD,), jnp.int32),
            pltpu.VMEM_SHARED((SC_SEGD,), jnp.float32),
            pltpu.SemaphoreType.DMA,
            pltpu.SemaphoreType.DMA,
        ),
    )
    iseg, vseg, pdiff, ppos = k1(index, train_highly_gene_idx, t16)

    k2 = pl.kernel(
        _stage2,
        out_type=(
            jax.ShapeDtypeStruct((N_TOT,), jnp.float32),
            jax.ShapeDtypeStruct((16,), jnp.int32),
        ),
        mesh=_mesh(),
        scratch_types=(
            pltpu.VMEM_SHARED((WSH,), jnp.float32),
            pltpu.VMEM((FILL,), jnp.float32),
            pltpu.VMEM((GSZ,), jnp.int32),
            pltpu.VMEM((GSZ,), jnp.int32),
            pltpu.VMEM((GSZ,), jnp.float32),
            pltpu.VMEM((GSZ,), jnp.float32),
            pltpu.VMEM((NW, 16), jnp.float32),
            pltpu.VMEM((NW, 16), jnp.int32),
            pltpu.VMEM((16,), jnp.int32),
            pltpu.VMEM((16,), jnp.int32),
            pltpu.SemaphoreType.DMA,
            pltpu.SemaphoreType.DMA,
            pltpu.SemaphoreType.DMA,
        ),
    )
    w, k16 = k2(iseg, vseg, pdiff, ppos, unnorm_index)
    return (w, w, thresh, k16[0])
